# baseline (speedup 1.0000x reference)
import functools

import jax
import jax.numpy as jnp
from jax.experimental import pallas as pl
from jax.experimental.pallas import tpu as pltpu

LAYER_SIZE = 128
NUM_CLASS = 100000
BATCH = 1024
PADDED_N = 100096
BLOCK_N = 2176
NBLOCKS = PADDED_N // BLOCK_N


def _matmul_kernel(x_ref, w_ref, b_ref, o_ref):
    xb = x_ref[...].astype(jnp.bfloat16)
    wb = w_ref[...].astype(jnp.bfloat16)
    acc = jax.lax.dot_general(
        xb, wb, (((1,), (1,)), ((), ())),
        preferred_element_type=jnp.float32,
    )
    o_ref[...] = acc + b_ref[...]


@functools.partial(jax.jit, static_argnames=())
def kernel(x, y, W, b):
    del y
    b_row = jnp.reshape(b, (1, NUM_CLASS))
    out = pl.pallas_call(
        _matmul_kernel,
        grid=(NBLOCKS,),
        in_specs=[
            pl.BlockSpec((BATCH, LAYER_SIZE), lambda i: (0, 0)),
            pl.BlockSpec((BLOCK_N, LAYER_SIZE), lambda i: (i, 0)),
            pl.BlockSpec((1, BLOCK_N), lambda i: (0, i)),
        ],
        out_specs=pl.BlockSpec((BATCH, BLOCK_N), lambda i: (0, i)),
        out_shape=jax.ShapeDtypeStruct((BATCH, PADDED_N), jnp.float32),
        compiler_params=pltpu.CompilerParams(
            dimension_semantics=("arbitrary",),
        ),
    )(x, W, b_row)
    return out[:, :NUM_CLASS]
